# baseline (speedup 1.0000x reference)
import functools

import jax
import jax.numpy as jnp
from jax import lax
from jax.experimental import pallas as pl
from jax.experimental.pallas import tpu as pltpu
from jax.experimental.pallas import tpu_sc as plsc


def _emb_call(bsz, seq, n, d, bpc):
    nc, ns = 2, 16
    nw = nc * ns
    b_per_w = bsz // nw
    chunk = bpc * seq
    per_w = b_per_w * seq
    n_chunks = b_per_w // bpc
    assert n_chunks % 2 == 0 and b_per_w % bpc == 0
    n_groups = n_chunks // 2
    seqp = 64
    seqx = 128
    mesh = plsc.VectorSubcoreMesh(core_axis_name="c", subcore_axis_name="s")

    @functools.partial(
        pl.kernel,
        out_type=jax.ShapeDtypeStruct((bsz, seq, d), jnp.float32),
        mesh=mesh,
        scratch_types=[
            pltpu.VMEM((b_per_w, seqp), jnp.int32),
            pltpu.VMEM((bpc, seqp, d), jnp.float32),
            pltpu.VMEM((bpc, seqp, d), jnp.float32),
            pltpu.SemaphoreType.DMA,
            pltpu.SemaphoreType.DMA,
            pltpu.SemaphoreType.DMA,
            pltpu.SemaphoreType.DMA,
        ],
        compiler_params=pltpu.CompilerParams(use_tc_tiling_on_sc=False),
    )
    def emb(x_hbm, table_hbm, out3_hbm, idx_v, rows0, rows1, g0, g1, w0, w1):
        wid = lax.axis_index("s") * nc + lax.axis_index("c")
        base = wid * b_per_w
        rows = (rows0, rows1)
        gsem = (g0, g1)
        wsem = (w0, w1)

        pltpu.sync_copy(x_hbm.at[pl.ds(wid * b_per_w, b_per_w), pl.ds(0, seqp)], idx_v)

        def gather(i, b):
            for j in range(bpc):
                pltpu.async_copy(
                    table_hbm.at[idx_v.at[i * bpc + j]],
                    rows[b].at[j],
                    gsem[b],
                )

        def put(i, b):
            pltpu.async_copy(rows[b].at[:, pl.ds(0, seq)], out3_hbm.at[pl.ds(base + i * bpc, bpc)], wsem[b])

        def wait_gather(b):
            for j in range(bpc):
                pltpu.make_async_copy(
                    table_hbm.at[idx_v.at[0]], rows[b].at[j], gsem[b]
                ).wait()

        def wait_put(b):
            pltpu.make_async_copy(rows[b].at[:, pl.ds(0, seq)], out3_hbm.at[pl.ds(0, bpc)], wsem[b]).wait()

        gather(0, 0)

        def group(g, carry):
            i0 = g * 2
            wait_gather(0)

            @pl.when(g > 0)
            def _():
                wait_put(1)

            gather(i0 + 1, 1)
            put(i0, 0)
            wait_gather(1)
            wait_put(0)

            @pl.when(g < n_groups - 1)
            def _():
                gather(i0 + 2, 0)

            put(i0 + 1, 1)
            return carry

        lax.fori_loop(0, n_groups, group, 0)
        wait_put(1)

    return emb


def kernel(x, weight):
    b, s = x.shape
    _, d = weight.shape
    xp = jnp.pad(x, ((0, 0), (0, 128 - s)))
    return _emb_call(b, s, b * s, d, bpc=16)(xp, weight)


# speedup vs baseline: 3.2412x; 3.2412x over previous
import functools

import jax
import jax.numpy as jnp
from jax import lax
from jax.experimental import pallas as pl
from jax.experimental.pallas import tpu as pltpu
from jax.experimental.pallas import tpu_sc as plsc


def _emb_call(bsz, seq, n, d, bpc):
    nc, ns = 2, 16
    nw = nc * ns
    b_per_w = bsz // nw
    chunk = bpc * seq
    per_w = b_per_w * seq
    n_chunks = b_per_w // bpc
    assert n_chunks % 2 == 0 and b_per_w % bpc == 0
    n_groups = n_chunks // 2
    seqp = (seq + 7) // 8 * 8
    seqx = 128
    mesh = plsc.VectorSubcoreMesh(core_axis_name="c", subcore_axis_name="s")

    @functools.partial(
        pl.kernel,
        out_type=jax.ShapeDtypeStruct((bsz, seq, d), jnp.float32),
        mesh=mesh,
        scratch_types=[
            pltpu.VMEM((b_per_w, seqp), jnp.int32),
            pltpu.VMEM((bpc, seqp, d), jnp.float32),
            pltpu.VMEM((bpc, seqp, d), jnp.float32),
            pltpu.SemaphoreType.DMA,
            pltpu.SemaphoreType.DMA,
            pltpu.SemaphoreType.DMA,
            pltpu.SemaphoreType.DMA,
        ],
        compiler_params=pltpu.CompilerParams(use_tc_tiling_on_sc=False),
    )
    def emb(x_hbm, table_hbm, out3_hbm, idx_v, rows0, rows1, g0, g1, w0, w1):
        wid = lax.axis_index("s") * nc + lax.axis_index("c")
        base = wid * b_per_w
        rows = (rows0, rows1)
        gsem = (g0, g1)
        wsem = (w0, w1)

        pltpu.sync_copy(x_hbm.at[pl.ds(wid * b_per_w, b_per_w), pl.ds(0, seqp)], idx_v)

        def gather(i, b):
            for j in range(bpc):
                pltpu.async_copy(
                    table_hbm.at[idx_v.at[i * bpc + j]],
                    rows[b].at[j],
                    gsem[b],
                )

        def put(i, b):
            pltpu.async_copy(rows[b].at[:, pl.ds(0, seq)], out3_hbm.at[pl.ds(base + i * bpc, bpc)], wsem[b])

        def wait_gather(b):
            for j in range(bpc):
                pltpu.make_async_copy(
                    table_hbm.at[idx_v.at[0]], rows[b].at[j], gsem[b]
                ).wait()

        def wait_put(b):
            pltpu.make_async_copy(rows[b].at[:, pl.ds(0, seq)], out3_hbm.at[pl.ds(0, bpc)], wsem[b]).wait()

        gather(0, 0)

        def group(g, carry):
            i0 = g * 2
            wait_gather(0)

            @pl.when(g > 0)
            def _():
                wait_put(1)

            gather(i0 + 1, 1)
            put(i0, 0)
            wait_gather(1)
            wait_put(0)

            @pl.when(g < n_groups - 1)
            def _():
                gather(i0 + 2, 0)

            put(i0 + 1, 1)
            return carry

        lax.fori_loop(0, n_groups, group, 0)
        wait_put(1)

    return emb


def kernel(x, weight):
    b, s = x.shape
    _, d = weight.shape
    xp = jnp.concatenate([x, x[:, : 128 - s]], axis=1)
    return _emb_call(b, s, b * s, d, bpc=16)(xp, weight)


# speedup vs baseline: 3.2529x; 1.0036x over previous
import functools

import jax
import jax.numpy as jnp
from jax import lax
from jax.experimental import pallas as pl
from jax.experimental.pallas import tpu as pltpu
from jax.experimental.pallas import tpu_sc as plsc


def _emb_call(bsz, seq, n, d, bpc):
    nc, ns = 2, 16
    nw = nc * ns
    b_per_w = bsz // nw
    chunk = bpc * seq
    per_w = b_per_w * seq
    n_chunks = b_per_w // bpc
    assert n_chunks % 2 == 0 and b_per_w % bpc == 0
    n_groups = n_chunks // 2
    seqp = (seq + 7) // 8 * 8
    seqx = 128
    mesh = plsc.VectorSubcoreMesh(core_axis_name="c", subcore_axis_name="s")

    @functools.partial(
        pl.kernel,
        out_type=jax.ShapeDtypeStruct((bsz, seq, d), jnp.float32),
        mesh=mesh,
        scratch_types=[
            pltpu.VMEM((b_per_w, seqp), jnp.float32),
            pltpu.VMEM((b_per_w, seqp), jnp.int32),
            pltpu.VMEM((bpc, seqp, d), jnp.float32),
            pltpu.VMEM((bpc, seqp, d), jnp.float32),
            pltpu.SemaphoreType.DMA,
            pltpu.SemaphoreType.DMA,
            pltpu.SemaphoreType.DMA,
            pltpu.SemaphoreType.DMA,
        ],
        compiler_params=pltpu.CompilerParams(
            use_tc_tiling_on_sc=False, needs_layout_passes=False
        ),
    )
    def emb(x_hbm, table_hbm, out3_hbm, idx_vf, idx_v, rows0, rows1, g0, g1, w0, w1):
        wid = lax.axis_index("s") * nc + lax.axis_index("c")
        base = wid * b_per_w
        rows = (rows0, rows1)
        gsem = (g0, g1)
        wsem = (w0, w1)

        pltpu.sync_copy(x_hbm.at[pl.ds(wid * b_per_w, b_per_w), pl.ds(0, seqp)], idx_vf)

        offs = tuple(range(0, seqp - 15, 16)) + ((seqp - 16,) if seqp % 16 else ())

        def cvt(r, carry):
            for c in offs:
                idx_v[r, pl.ds(c, 16)] = plsc.bitcast(idx_vf[r, pl.ds(c, 16)], jnp.int32)
            return carry

        lax.fori_loop(0, b_per_w, cvt, 0)

        def gather(i, b):
            for j in range(bpc):
                pltpu.async_copy(
                    table_hbm.at[idx_v.at[i * bpc + j]],
                    rows[b].at[j],
                    gsem[b],
                )

        def put(i, b):
            pltpu.async_copy(rows[b].at[:, pl.ds(0, seq)], out3_hbm.at[pl.ds(base + i * bpc, bpc)], wsem[b])

        def wait_gather(b):
            for j in range(bpc):
                pltpu.make_async_copy(
                    table_hbm.at[idx_v.at[0]], rows[b].at[j], gsem[b]
                ).wait()

        def wait_put(b):
            pltpu.make_async_copy(rows[b].at[:, pl.ds(0, seq)], out3_hbm.at[pl.ds(0, bpc)], wsem[b]).wait()

        gather(0, 0)

        def group(g, carry):
            i0 = g * 2
            wait_gather(0)

            @pl.when(g > 0)
            def _():
                wait_put(1)

            gather(i0 + 1, 1)
            put(i0, 0)
            wait_gather(1)
            wait_put(0)

            @pl.when(g < n_groups - 1)
            def _():
                gather(i0 + 2, 0)

            put(i0 + 1, 1)
            return carry

        lax.fori_loop(0, n_groups, group, 0)
        wait_put(1)

    return emb


def kernel(x, weight):
    b, s = x.shape
    _, d = weight.shape
    sp = (s + 7) // 8 * 8
    xp = jnp.concatenate([x, x[:, : sp - s]], axis=1)
    xf = jax.lax.bitcast_convert_type(xp, jnp.float32)
    return _emb_call(b, s, b * s, d, bpc=16)(xf, weight)


# speedup vs baseline: 3.2710x; 1.0056x over previous
import functools

import jax
import jax.numpy as jnp
from jax import lax
from jax.experimental import pallas as pl
from jax.experimental.pallas import tpu as pltpu
from jax.experimental.pallas import tpu_sc as plsc


def _emb_call(bsz, seq, n, d, bpc):
    nc, ns = 2, 16
    nw = nc * ns
    b_per_w = bsz // nw
    n_chunks = b_per_w // bpc
    assert n_chunks % 2 == 0 and b_per_w % bpc == 0
    n_groups = n_chunks // 2
    mesh = plsc.VectorSubcoreMesh(core_axis_name="c", subcore_axis_name="s")

    @functools.partial(
        pl.kernel,
        out_type=jax.ShapeDtypeStruct((bsz, seq, d), jnp.float32),
        mesh=mesh,
        scratch_types=[
            pltpu.VMEM((b_per_w, seq), jnp.int32),
            pltpu.VMEM((bpc, seq, d), jnp.float32),
            pltpu.VMEM((bpc, seq, d), jnp.float32),
            pltpu.SemaphoreType.DMA,
            pltpu.SemaphoreType.DMA,
            pltpu.SemaphoreType.DMA,
            pltpu.SemaphoreType.DMA,
        ],
        compiler_params=pltpu.CompilerParams(use_tc_tiling_on_sc=False),
    )
    def emb(x_hbm, table_hbm, out3_hbm, idx_v, rows0, rows1, g0, g1, w0, w1):
        wid = lax.axis_index("s") * nc + lax.axis_index("c")
        base = wid * b_per_w
        rows = (rows0, rows1)
        gsem = (g0, g1)
        wsem = (w0, w1)

        pltpu.sync_copy(x_hbm.at[pl.ds(wid * b_per_w, b_per_w)], idx_v)

        def gather(i, b):
            for j in range(bpc):
                pltpu.async_copy(
                    table_hbm.at[idx_v.at[i * bpc + j]],
                    rows[b].at[j],
                    gsem[b],
                )

        def put(i, b):
            pltpu.async_copy(rows[b], out3_hbm.at[pl.ds(base + i * bpc, bpc)], wsem[b])

        def wait_gather(b):
            for j in range(bpc):
                pltpu.make_async_copy(
                    table_hbm.at[idx_v.at[0]], rows[b].at[j], gsem[b]
                ).wait()

        def wait_put(b):
            pltpu.make_async_copy(rows[b], out3_hbm.at[pl.ds(0, bpc)], wsem[b]).wait()

        gather(0, 0)

        def group(g, carry):
            i0 = g * 2
            wait_gather(0)

            @pl.when(g > 0)
            def _():
                wait_put(1)

            gather(i0 + 1, 1)
            put(i0, 0)
            wait_gather(1)
            wait_put(0)

            @pl.when(g < n_groups - 1)
            def _():
                gather(i0 + 2, 0)

            put(i0 + 1, 1)
            return carry

        lax.fori_loop(0, n_groups, group, 0)
        wait_put(1)

    return emb


def kernel(x, weight):
    b, s = x.shape
    _, d = weight.shape
    return _emb_call(b, s, b * s, d, bpc=16)(x, weight)
